# baseline (speedup 1.0000x reference)
import functools

import jax
import jax.numpy as jnp
from jax import lax
from jax.experimental import pallas as pl
from jax.experimental.pallas import tpu as pltpu
from jax.experimental.pallas import tpu_sc as plsc

_NC = 2
_NS = 16
_NW = _NC * _NS
_CHUNK = 128


@functools.cache
def _make_fused(n_ch, rows_w, d):
    mesh = plsc.VectorSubcoreMesh(core_axis_name="c", subcore_axis_name="s")

    @functools.partial(
        pl.kernel,
        out_type=(
            jax.ShapeDtypeStruct((_NW, n_ch, _CHUNK), jnp.int32),
            jax.ShapeDtypeStruct((_NW * rows_w, d), jnp.float32),
        ),
        mesh=mesh,
        scratch_types=[
            pltpu.VMEM((n_ch, _CHUNK), jnp.int32),
            pltpu.VMEM((n_ch, _CHUNK), jnp.int32),
            pltpu.SemaphoreType.DMA,
            pltpu.SemaphoreType.DMA,
        ],
    )
    def fused_kernel(ids_hbm, table_hbm, feat_hbm, out_hbm, feat_out_hbm,
                     idx_v, vals_v, sem, fsem):
        wid = lax.axis_index("s") * _NC + lax.axis_index("c")
        rbase = wid * rows_w
        fcopy = pltpu.async_copy(
            feat_hbm.at[pl.ds(rbase, rows_w)],
            feat_out_hbm.at[pl.ds(rbase, rows_w)],
            fsem,
        )
        pltpu.sync_copy(ids_hbm.at[wid], idx_v)
        copies = [
            pltpu.async_copy(table_hbm.at[idx_v.at[j]], vals_v.at[j], sem)
            for j in range(n_ch)
        ]
        for c in copies:
            c.wait()
        pltpu.sync_copy(vals_v, out_hbm.at[wid])
        fcopy.wait()

    return fused_kernel


def kernel(input_ids, features, lookup_table):
    batch = input_ids.shape[0]
    n_ch = batch // (_NW * _CHUNK)
    rows, d = features.shape
    ids = input_ids.astype(jnp.int32).reshape(_NW, n_ch, _CHUNK)
    out, feat_out = _make_fused(n_ch, rows // _NW, d)(ids, lookup_table, features)
    return (out.reshape(batch), feat_out)


# speedup vs baseline: 11.5578x; 11.5578x over previous
import functools

import jax
import jax.numpy as jnp
from jax import lax
from jax.experimental import pallas as pl
from jax.experimental.pallas import tpu as pltpu
from jax.experimental.pallas import tpu_sc as plsc

_NC = 2
_NS = 16
_NW = _NC * _NS
_CHUNK = 128


@functools.cache
def _make_gather(n_ch):
    mesh = plsc.VectorSubcoreMesh(core_axis_name="c", subcore_axis_name="s")
    half = n_ch // 2

    @functools.partial(
        pl.kernel,
        out_type=jax.ShapeDtypeStruct((_NW, n_ch, _CHUNK), jnp.int32),
        mesh=mesh,
        scratch_types=[
            pltpu.VMEM((n_ch, _CHUNK), jnp.int32),
            pltpu.VMEM((n_ch, _CHUNK), jnp.int32),
            pltpu.SemaphoreType.DMA,
            pltpu.SemaphoreType.DMA,
        ],
    )
    def gather_kernel(ids_hbm, table_hbm, out_hbm, idx_v, vals_v, sem, isem):
        wid = lax.axis_index("s") * _NC + lax.axis_index("c")
        pltpu.sync_copy(ids_hbm.at[wid, pl.ds(0, half)], idx_v.at[pl.ds(0, half)])
        copies = [
            pltpu.async_copy(table_hbm.at[idx_v.at[j]], vals_v.at[j], sem)
            for j in range(half)
        ]
        icopy = pltpu.async_copy(
            ids_hbm.at[wid, pl.ds(half, n_ch - half)],
            idx_v.at[pl.ds(half, n_ch - half)], isem,
        )
        icopy.wait()
        copies += [
            pltpu.async_copy(table_hbm.at[idx_v.at[j]], vals_v.at[j], sem)
            for j in range(half, n_ch)
        ]
        for c in copies:
            c.wait()
        pltpu.sync_copy(vals_v, out_hbm.at[wid])

    return gather_kernel


def kernel(input_ids, features, lookup_table):
    batch = input_ids.shape[0]
    n_ch = batch // (_NW * _CHUNK)
    ids = input_ids.astype(jnp.int32).reshape(_NW, n_ch, _CHUNK)
    out = _make_gather(n_ch)(ids, lookup_table)
    return (out.reshape(batch), features)


# speedup vs baseline: 11.6213x; 1.0055x over previous
import functools

import jax
import jax.numpy as jnp
from jax import lax
from jax.experimental import pallas as pl
from jax.experimental.pallas import tpu as pltpu
from jax.experimental.pallas import tpu_sc as plsc

_NC = 2
_NS = 16
_NW = _NC * _NS
_CHUNK = 128


@functools.cache
def _make_gather(n_ch):
    mesh = plsc.VectorSubcoreMesh(core_axis_name="c", subcore_axis_name="s")

    @functools.partial(
        pl.kernel,
        out_type=jax.ShapeDtypeStruct((_NW, n_ch, _CHUNK), jnp.int32),
        mesh=mesh,
        scratch_types=[
            pltpu.VMEM((n_ch, _CHUNK), jnp.int32),
            pltpu.VMEM((n_ch, _CHUNK), jnp.int32),
            pltpu.SemaphoreType.DMA,
            pltpu.SemaphoreType.DMA,
        ],
    )
    def gather_kernel(ids_hbm, table_hbm, out_hbm, idx_v, vals_v, sem, osem):
        wid = lax.axis_index("s") * _NC + lax.axis_index("c")
        pltpu.sync_copy(ids_hbm.at[wid], idx_v)
        copies = [
            pltpu.async_copy(table_hbm.at[idx_v.at[j]], vals_v.at[j], sem)
            for j in range(n_ch)
        ]
        ocopies = []
        for j in range(n_ch):
            copies[j].wait()
            ocopies.append(
                pltpu.async_copy(vals_v.at[j], out_hbm.at[wid, j], osem)
            )
        for c in ocopies:
            c.wait()

    return gather_kernel


def kernel(input_ids, features, lookup_table):
    batch = input_ids.shape[0]
    n_ch = batch // (_NW * _CHUNK)
    ids = input_ids.astype(jnp.int32).reshape(_NW, n_ch, _CHUNK)
    out = _make_gather(n_ch)(ids, lookup_table)
    return (out.reshape(batch), features)


# speedup vs baseline: 11.6276x; 1.0005x over previous
import functools

import jax
import jax.numpy as jnp
from jax import lax
from jax.experimental import pallas as pl
from jax.experimental.pallas import tpu as pltpu
from jax.experimental.pallas import tpu_sc as plsc

_NC = 2
_NS = 16
_NW = _NC * _NS
_CHUNK = 128


@functools.cache
def _make_gather(batch):
    b_w = batch // _NW
    n_ch = b_w // _CHUNK
    mesh = plsc.VectorSubcoreMesh(core_axis_name="c", subcore_axis_name="s")

    @functools.partial(
        pl.kernel,
        out_type=jax.ShapeDtypeStruct((batch,), jnp.int32),
        mesh=mesh,
        scratch_types=[
            pltpu.VMEM((b_w,), jnp.int32),
            pltpu.VMEM((b_w,), jnp.int32),
            pltpu.SemaphoreType.DMA,
            pltpu.SemaphoreType.DMA,
        ],
    )
    def gather_kernel(ids_hbm, table_hbm, out_hbm, idx_v, vals_v, sem, osem):
        wid = lax.axis_index("s") * _NC + lax.axis_index("c")
        base = wid * b_w
        pltpu.sync_copy(ids_hbm.at[pl.ds(base, b_w)], idx_v)
        copies = [
            pltpu.async_copy(
                table_hbm.at[idx_v.at[pl.ds(j * _CHUNK, _CHUNK)]],
                vals_v.at[pl.ds(j * _CHUNK, _CHUNK)],
                sem,
            )
            for j in range(n_ch)
        ]
        ocopies = []
        for j in range(n_ch):
            copies[j].wait()
            ocopies.append(
                pltpu.async_copy(
                    vals_v.at[pl.ds(j * _CHUNK, _CHUNK)],
                    out_hbm.at[pl.ds(base + j * _CHUNK, _CHUNK)],
                    osem,
                )
            )
        for c in ocopies:
            c.wait()

    return gather_kernel


def kernel(input_ids, features, lookup_table):
    batch = input_ids.shape[0]
    ids = input_ids.astype(jnp.int32)
    out = _make_gather(batch)(ids, lookup_table)
    return (out, features)


# speedup vs baseline: 11.6621x; 1.0030x over previous
import functools

import jax
import jax.numpy as jnp
from jax import lax
from jax.experimental import pallas as pl
from jax.experimental.pallas import tpu as pltpu
from jax.experimental.pallas import tpu_sc as plsc

_NC = 2
_NS = 16
_NW = _NC * _NS
_CHUNK = 128


@functools.cache
def _make_gather(batch):
    b_w = batch // _NW
    n_ch = b_w // _CHUNK
    mesh = plsc.VectorSubcoreMesh(core_axis_name="c", subcore_axis_name="s")

    @functools.partial(
        pl.kernel,
        out_type=jax.ShapeDtypeStruct((batch,), jnp.int32),
        mesh=mesh,
        scratch_types=[
            pltpu.VMEM((b_w,), jnp.int32),
            pltpu.VMEM((b_w,), jnp.int32),
            pltpu.SemaphoreType.DMA,
            pltpu.SemaphoreType.DMA,
        ],
    )
    def gather_kernel(ids_hbm, table_hbm, out_hbm, idx_v, vals_v, sem, osem):
        wid = lax.axis_index("s") * _NC + lax.axis_index("c")
        base = wid * b_w
        pltpu.sync_copy(ids_hbm.at[pl.ds(base, b_w)], idx_v)
        copies = [
            pltpu.async_copy(
                table_hbm.at[idx_v.at[pl.ds(j * _CHUNK, _CHUNK)]],
                vals_v.at[pl.ds(j * _CHUNK, _CHUNK)],
                sem,
            )
            for j in range(n_ch)
        ]
        ocopies = []
        for j in range(n_ch):
            copies[j].wait()
            ocopies.append(
                pltpu.async_copy(
                    vals_v.at[pl.ds(j * _CHUNK, _CHUNK)],
                    out_hbm.at[pl.ds(base + j * _CHUNK, _CHUNK)],
                    osem,
                )
            )
        for c in ocopies:
            c.wait()

    return gather_kernel


def kernel(input_ids, features, lookup_table):
    batch = input_ids.shape[0]
    ids = input_ids.astype(jnp.int32)
    feat_out = jnp.copy(features)
    feat_out, ids = lax.optimization_barrier((feat_out, ids))
    out = _make_gather(batch)(ids, lookup_table)
    return (out, feat_out)
